# baseline (speedup 1.0000x reference)
import functools

import jax
import jax.numpy as jnp
from jax import lax
from jax.experimental import pallas as pl
from jax.experimental.pallas import tpu as pltpu
from jax.experimental.pallas import tpu_sc as plsc

_B = 16384
_NC = 2
_NS = 16
_NW = _NC * _NS
_BPW = _B // _NW
_CH = 128
_NCHUNK = _BPW // _CH

_BR = 2048
_NB = _B // _BR


def _sc_gather_body(movie_tab, mid, out_m, idx_v, rows_sh,
                    sem0, sem1, sem2, sem3):
    wid = lax.axis_index("s") * _NC + lax.axis_index("c")
    sid = lax.axis_index("s")
    base = wid * _BPW
    sbase = sid * _BPW
    pltpu.sync_copy(mid.at[wid], idx_v)
    lanes = lax.broadcasted_iota(jnp.int32, (16,), 0)
    sems = (sem0, sem1, sem2, sem3)

    def fire(k, carry):
        v = idx_v[pl.ds(k * 16, 16)]
        for j in range(16):
            idx = jnp.sum(jnp.where(lanes == j, v, 0))
            pltpu.async_copy(movie_tab.at[pl.ds(idx, 1)],
                             rows_sh.at[pl.ds(sbase + k * 16 + j, 1)],
                             sems[j % 4])
        return carry

    lax.fori_loop(0, _BPW // 16, fire, 0)
    for b in range(4):
        pltpu.make_async_copy(movie_tab.at[pl.ds(0, _BPW // 4)],
                              rows_sh.at[pl.ds(sbase + b * (_BPW // 4),
                                               _BPW // 4)],
                              sems[b]).wait()
    pltpu.sync_copy(rows_sh.at[pl.ds(sbase, _BPW)],
                    out_m.at[pl.ds(base, _BPW)])


@functools.cache
def _sc_gather():
    return pl.kernel(
        _sc_gather_body,
        out_type=jax.ShapeDtypeStruct((_B, 64), jnp.float32),
        mesh=plsc.VectorSubcoreMesh(core_axis_name="c", subcore_axis_name="s"),
        scratch_types=[
            pltpu.VMEM((_BPW,), jnp.int32),
            pltpu.VMEM_SHARED((_NS * _BPW, 64), jnp.float32),
            pltpu.SemaphoreType.DMA,
            pltpu.SemaphoreType.DMA,
            pltpu.SemaphoreType.DMA,
            pltpu.SemaphoreType.DMA,
        ],
        compiler_params=pltpu.CompilerParams(needs_layout_passes=False),
    )


def _mlp_body(m, lid, cid, r, lang_tab, ct_tab, w1m, w1l, w1c, w1r, b1, g1, be1,
              w2, b2, g2, be2, w3, b3,
              out, h1_buf, h2_buf, s1, ss1, s2, ss2):
    p = pl.program_id(0)
    i = pl.program_id(1)
    inv_b = 1.0 / _B
    eps = 1e-5

    @pl.when(p == 0)
    def _phase0():
        @pl.when(i == 0)
        def _init():
            s1[...] = jnp.zeros_like(s1)
            ss1[...] = jnp.zeros_like(ss1)

        lrow = lid[...].reshape(1, _BR)
        crow = cid[...].reshape(1, _BR)
        oh_l = (lrow == lax.broadcasted_iota(jnp.int32, (128, _BR), 0)
                ).astype(jnp.float32)
        oh_c = (crow == lax.broadcasted_iota(jnp.int32, (16, _BR), 0)
                ).astype(jnp.float32)
        dn = (((0,), (0,)), ((), ()))
        l_emb = lax.dot_general(oh_l, lang_tab[...], dn,
                                preferred_element_type=jnp.float32)
        c_emb = lax.dot_general(oh_c, ct_tab[...], dn,
                                preferred_element_type=jnp.float32)
        h = (jnp.dot(m[...], w1m[...], preferred_element_type=jnp.float32)
             + jnp.dot(l_emb, w1l[...], preferred_element_type=jnp.float32)
             + jnp.dot(c_emb, w1c[...], preferred_element_type=jnp.float32)
             + jnp.dot(r[...], w1r[...], preferred_element_type=jnp.float32)
             + b1[...])
        h1_buf[pl.ds(i * _BR, _BR), :] = h
        s1[...] += jnp.sum(h, axis=0, keepdims=True)
        ss1[...] += jnp.sum(h * h, axis=0, keepdims=True)

    @pl.when(p == 1)
    def _phase1():
        @pl.when(i == 0)
        def _init():
            s2[...] = jnp.zeros_like(s2)
            ss2[...] = jnp.zeros_like(ss2)

        mean = s1[...] * inv_b
        var = ss1[...] * inv_b - mean * mean
        scale = lax.rsqrt(var + eps) * g1[...]
        shift = be1[...] - mean * scale
        a = jnp.maximum(h1_buf[pl.ds(i * _BR, _BR), :] * scale + shift, 0.0)
        h2 = jnp.dot(a, w2[...], preferred_element_type=jnp.float32) + b2[...]
        h2_buf[pl.ds(i * _BR, _BR), :] = h2
        s2[...] += jnp.sum(h2, axis=0, keepdims=True)
        ss2[...] += jnp.sum(h2 * h2, axis=0, keepdims=True)

    @pl.when(p == 2)
    def _phase2():
        mean = s2[...] * inv_b
        var = ss2[...] * inv_b - mean * mean
        scale = lax.rsqrt(var + eps) * g2[...]
        shift = be2[...] - mean * scale
        a = jnp.maximum(h2_buf[pl.ds(i * _BR, _BR), :] * scale + shift, 0.0)
        out[...] = jnp.dot(a, w3[...], preferred_element_type=jnp.float32) + b3[...]


def _full(shape):
    return pl.BlockSpec(shape, lambda p, i: (0, 0))


_mlp = pl.pallas_call(
    _mlp_body,
    grid=(3, _NB),
    in_specs=[
        pl.BlockSpec((_BR, 64), lambda p, i: (jnp.where(p == 0, i, 0), 0)),
        pl.BlockSpec((1, 1, _BR), lambda p, i: (jnp.where(p == 0, i, 0), 0, 0)),
        pl.BlockSpec((1, 1, _BR), lambda p, i: (jnp.where(p == 0, i, 0), 0, 0)),
        pl.BlockSpec((_BR, 32), lambda p, i: (jnp.where(p == 0, i, 0), 0)),
        _full((128, 16)),
        _full((16, 8)),
        _full((64, 256)),
        _full((16, 256)),
        _full((8, 256)),
        _full((32, 256)),
        _full((1, 256)),
        _full((1, 256)),
        _full((1, 256)),
        _full((256, 128)),
        _full((1, 128)),
        _full((1, 128)),
        _full((1, 128)),
        _full((128, 128)),
        _full((1, 128)),
    ],
    out_specs=pl.BlockSpec((_BR, 128), lambda p, i: (i, 0)),
    out_shape=jax.ShapeDtypeStruct((_B, 128), jnp.float32),
    scratch_shapes=[
        pltpu.VMEM((_B, 256), jnp.float32),
        pltpu.VMEM((_B, 128), jnp.float32),
        pltpu.VMEM((1, 256), jnp.float32),
        pltpu.VMEM((1, 256), jnp.float32),
        pltpu.VMEM((1, 128), jnp.float32),
        pltpu.VMEM((1, 128), jnp.float32),
    ],
    compiler_params=pltpu.CompilerParams(
        dimension_semantics=("arbitrary", "arbitrary"),
    ),
)


@jax.jit
def kernel(movie_id, movie_continuous, movie_language, movie_content_type,
           movie_genres, movie_emb, lang_emb, ct_emb,
           W1, b1, g1, beta1, W2, b2, g2, beta2, W3, b3):
    mid = movie_id.astype(jnp.int32).reshape(_NW, _BPW)
    lid = movie_language.astype(jnp.int32).reshape(_NB, 1, _BR)
    cid = movie_content_type.astype(jnp.int32).reshape(_NB, 1, _BR)
    lang_tab = jnp.pad(lang_emb, ((0, 27), (0, 0)))
    ct_tab = jnp.pad(ct_emb, ((0, 5), (0, 0)))
    rest = jnp.concatenate([movie_continuous, movie_genres], axis=1)
    rest = jnp.pad(rest, ((0, 0), (0, 9)))
    w1m = W1[0:64]
    w1l = W1[64:80]
    w1c = W1[80:88]
    w1r = jnp.pad(W1[88:111], ((0, 9), (0, 0)))

    m = _sc_gather()(movie_emb, mid)

    return _mlp(m, lid, cid, rest, lang_tab, ct_tab, w1m, w1l, w1c, w1r,
                b1.reshape(1, -1), g1.reshape(1, -1), beta1.reshape(1, -1),
                W2, b2.reshape(1, -1), g2.reshape(1, -1), beta2.reshape(1, -1),
                W3, b3.reshape(1, -1))


# speedup vs baseline: 1.0781x; 1.0781x over previous
import functools

import jax
import jax.numpy as jnp
from jax import lax
from jax.experimental import pallas as pl
from jax.experimental.pallas import tpu as pltpu
from jax.experimental.pallas import tpu_sc as plsc

_B = 16384
_NC = 2
_NS = 16
_NW = _NC * _NS
_BPW = _B // _NW
_CH = 128
_NCHUNK = _BPW // _CH

_BR = 4096
_NB = _B // _BR


def _sc_gather_body(movie_tab, mid, out_m, idx_v, rows_v,
                    sem0, sem1, sem2, sem3):
    wid = lax.axis_index("s") * _NC + lax.axis_index("c")
    base = wid * _BPW
    pltpu.sync_copy(mid.at[wid], idx_v)
    lanes = lax.broadcasted_iota(jnp.int32, (16,), 0)
    sems = (sem0, sem1, sem2, sem3)

    def fire(k, carry):
        v = idx_v[pl.ds(k * 16, 16)]
        for j in range(16):
            idx = jnp.sum(jnp.where(lanes == j, v, 0))
            pltpu.async_copy(movie_tab.at[pl.ds(idx, 1)],
                             rows_v.at[pl.ds(k * 16 + j, 1)], sems[j % 4])
        return carry

    lax.fori_loop(0, _BPW // 16, fire, 0)
    for b in range(4):
        pltpu.make_async_copy(movie_tab.at[pl.ds(0, _BPW // 4)],
                              rows_v.at[pl.ds(b * (_BPW // 4), _BPW // 4)],
                              sems[b]).wait()
    pltpu.sync_copy(rows_v, out_m.at[pl.ds(base, _BPW)])


@functools.cache
def _sc_gather():
    return pl.kernel(
        _sc_gather_body,
        out_type=jax.ShapeDtypeStruct((_B, 64), jnp.float32),
        mesh=plsc.VectorSubcoreMesh(core_axis_name="c", subcore_axis_name="s"),
        scratch_types=[
            pltpu.VMEM((_BPW,), jnp.int32),
            pltpu.VMEM((_BPW, 64), jnp.float32),
            pltpu.SemaphoreType.DMA,
            pltpu.SemaphoreType.DMA,
            pltpu.SemaphoreType.DMA,
            pltpu.SemaphoreType.DMA,
        ],
        compiler_params=pltpu.CompilerParams(needs_layout_passes=False),
    )


def _mlp_body(m, lid, cid, r, lang_tab, ct_tab, w1m, w1l, w1c, w1r, b1, g1, be1,
              w2, b2, g2, be2, w3, b3,
              out, h1_buf, h2_buf, s1, ss1, s2, ss2):
    p = pl.program_id(0)
    i = pl.program_id(1)
    inv_b = 1.0 / _B
    eps = 1e-5

    @pl.when(p == 0)
    def _phase0():
        @pl.when(i == 0)
        def _init():
            s1[...] = jnp.zeros_like(s1)
            ss1[...] = jnp.zeros_like(ss1)

        lrow = lid[...].reshape(1, _BR)
        crow = cid[...].reshape(1, _BR)
        oh_l = (lrow == lax.broadcasted_iota(jnp.int32, (128, _BR), 0)
                ).astype(jnp.float32)
        oh_c = (crow == lax.broadcasted_iota(jnp.int32, (16, _BR), 0)
                ).astype(jnp.float32)
        dn = (((0,), (0,)), ((), ()))
        l_emb = lax.dot_general(oh_l, lang_tab[...], dn,
                                preferred_element_type=jnp.float32)
        c_emb = lax.dot_general(oh_c, ct_tab[...], dn,
                                preferred_element_type=jnp.float32)
        h = (jnp.dot(m[...], w1m[...], preferred_element_type=jnp.float32)
             + jnp.dot(l_emb, w1l[...], preferred_element_type=jnp.float32)
             + jnp.dot(c_emb, w1c[...], preferred_element_type=jnp.float32)
             + jnp.dot(r[...], w1r[...], preferred_element_type=jnp.float32)
             + b1[...])
        h1_buf[pl.ds(i * _BR, _BR), :] = h
        s1[...] += jnp.sum(h, axis=0, keepdims=True)
        ss1[...] += jnp.sum(h * h, axis=0, keepdims=True)

    @pl.when(p == 1)
    def _phase1():
        @pl.when(i == 0)
        def _init():
            s2[...] = jnp.zeros_like(s2)
            ss2[...] = jnp.zeros_like(ss2)

        mean = s1[...] * inv_b
        var = ss1[...] * inv_b - mean * mean
        scale = lax.rsqrt(var + eps) * g1[...]
        shift = be1[...] - mean * scale
        a = jnp.maximum(h1_buf[pl.ds(i * _BR, _BR), :] * scale + shift, 0.0)
        h2 = jnp.dot(a, w2[...], preferred_element_type=jnp.float32) + b2[...]
        h2_buf[pl.ds(i * _BR, _BR), :] = h2
        s2[...] += jnp.sum(h2, axis=0, keepdims=True)
        ss2[...] += jnp.sum(h2 * h2, axis=0, keepdims=True)

    @pl.when(p == 2)
    def _phase2():
        mean = s2[...] * inv_b
        var = ss2[...] * inv_b - mean * mean
        scale = lax.rsqrt(var + eps) * g2[...]
        shift = be2[...] - mean * scale
        a = jnp.maximum(h2_buf[pl.ds(i * _BR, _BR), :] * scale + shift, 0.0)
        out[...] = jnp.dot(a, w3[...], preferred_element_type=jnp.float32) + b3[...]


def _full(shape):
    return pl.BlockSpec(shape, lambda p, i: (0, 0))


_mlp = pl.pallas_call(
    _mlp_body,
    grid=(3, _NB),
    in_specs=[
        pl.BlockSpec((_BR, 64), lambda p, i: (jnp.where(p == 0, i, 0), 0)),
        pl.BlockSpec((1, 1, _BR), lambda p, i: (jnp.where(p == 0, i, 0), 0, 0)),
        pl.BlockSpec((1, 1, _BR), lambda p, i: (jnp.where(p == 0, i, 0), 0, 0)),
        pl.BlockSpec((_BR, 32), lambda p, i: (jnp.where(p == 0, i, 0), 0)),
        _full((128, 16)),
        _full((16, 8)),
        _full((64, 256)),
        _full((16, 256)),
        _full((8, 256)),
        _full((32, 256)),
        _full((1, 256)),
        _full((1, 256)),
        _full((1, 256)),
        _full((256, 128)),
        _full((1, 128)),
        _full((1, 128)),
        _full((1, 128)),
        _full((128, 128)),
        _full((1, 128)),
    ],
    out_specs=pl.BlockSpec((_BR, 128), lambda p, i: (i, 0)),
    out_shape=jax.ShapeDtypeStruct((_B, 128), jnp.float32),
    scratch_shapes=[
        pltpu.VMEM((_B, 256), jnp.float32),
        pltpu.VMEM((_B, 128), jnp.float32),
        pltpu.VMEM((1, 256), jnp.float32),
        pltpu.VMEM((1, 256), jnp.float32),
        pltpu.VMEM((1, 128), jnp.float32),
        pltpu.VMEM((1, 128), jnp.float32),
    ],
    compiler_params=pltpu.CompilerParams(
        dimension_semantics=("arbitrary", "arbitrary"),
    ),
)


@jax.jit
def kernel(movie_id, movie_continuous, movie_language, movie_content_type,
           movie_genres, movie_emb, lang_emb, ct_emb,
           W1, b1, g1, beta1, W2, b2, g2, beta2, W3, b3):
    mid = movie_id.astype(jnp.int32).reshape(_NW, _BPW)
    lid = movie_language.astype(jnp.int32).reshape(_NB, 1, _BR)
    cid = movie_content_type.astype(jnp.int32).reshape(_NB, 1, _BR)
    lang_tab = jnp.pad(lang_emb, ((0, 27), (0, 0)))
    ct_tab = jnp.pad(ct_emb, ((0, 5), (0, 0)))
    rest = jnp.concatenate([movie_continuous, movie_genres], axis=1)
    rest = jnp.pad(rest, ((0, 0), (0, 9)))
    w1m = W1[0:64]
    w1l = W1[64:80]
    w1c = W1[80:88]
    w1r = jnp.pad(W1[88:111], ((0, 9), (0, 0)))

    m = _sc_gather()(movie_emb, mid)

    return _mlp(m, lid, cid, rest, lang_tab, ct_tab, w1m, w1l, w1c, w1r,
                b1.reshape(1, -1), g1.reshape(1, -1), beta1.reshape(1, -1),
                W2, b2.reshape(1, -1), g2.reshape(1, -1), beta2.reshape(1, -1),
                W3, b3.reshape(1, -1))
